# Initial kernel scaffold; baseline (speedup 1.0000x reference)
import jax
import jax.numpy as jnp
from jax.experimental import pallas as pl


def kernel(x, edge_index, sequence_feature, W0, b0, W1, b1, W2, b2, Ws, bs, Wl1, bl1, Wl2, bl2, Wl3, bl3):
    raise NotImplementedError("write your pallas kernel here")


# speedup vs baseline: 5.4049x; 5.4049x over previous
import functools

import jax
import jax.numpy as jnp
from jax import lax
from jax.experimental import pallas as pl
from jax.experimental.pallas import tpu as pltpu
from jax.experimental.pallas import tpu_sc as plsc

N = 10000
E = 320000
HID = 128
K = 5000

NC = 2
NS = 16
NW = NC * NS
PER_TEC = E // NW
CHUNK = 80
STEPS = PER_TEC // CHUNK

ROWS_LO = 624
ROWS_HI = N - 15 * ROWS_LO
ZROWS = 640

_mesh = plsc.VectorSubcoreMesh(core_axis_name="c", subcore_axis_name="s")


def _mp_pass(src, dst, table, zeros):
    d = table.shape[1]

    @functools.partial(
        pl.kernel,
        out_type=jax.ShapeDtypeStruct((NC, N, d), jnp.float32),
        mesh=_mesh,
        compiler_params=pltpu.CompilerParams(use_tc_tiling_on_sc=(d == HID)),
        scratch_types=[
            pltpu.VMEM_SHARED((N, d), jnp.float32),
            pltpu.VMEM((CHUNK,), jnp.int32),
            pltpu.VMEM((CHUNK,), jnp.int32),
            pltpu.VMEM((CHUNK, d), jnp.float32),
        ],
    )
    def k(src_hbm, dst_hbm, table_hbm, zeros_hbm, out_hbm, acc, sidx, didx, rows):
        cid = lax.axis_index("c")
        sid = lax.axis_index("s")
        base = sid * ROWS_LO

        @pl.when(sid < NS - 1)
        def _():
            pltpu.sync_copy(zeros_hbm.at[pl.ds(0, ROWS_LO)], acc.at[pl.ds(base, ROWS_LO)])

        @pl.when(sid == NS - 1)
        def _():
            pltpu.sync_copy(zeros_hbm.at[pl.ds(0, ROWS_HI)],
                            acc.at[pl.ds(15 * ROWS_LO, ROWS_HI)])

        plsc.subcore_barrier()

        e0 = (cid * NS + sid) * PER_TEC

        def body(i, carry):
            off = e0 + i * CHUNK
            pltpu.sync_copy(src_hbm.at[pl.ds(off, CHUNK)], sidx)
            pltpu.sync_copy(dst_hbm.at[pl.ds(off, CHUNK)], didx)
            pltpu.sync_copy(table_hbm.at[sidx], rows)
            pltpu.sync_copy(rows, acc.at[didx], add=True)
            return carry

        lax.fori_loop(0, STEPS, body, 0)
        plsc.subcore_barrier()

        @pl.when(sid < NS - 1)
        def _():
            pltpu.sync_copy(acc.at[pl.ds(base, ROWS_LO)],
                            out_hbm.at[cid, pl.ds(base, ROWS_LO)])

        @pl.when(sid == NS - 1)
        def _():
            pltpu.sync_copy(acc.at[pl.ds(15 * ROWS_LO, ROWS_HI)],
                            out_hbm.at[cid, pl.ds(15 * ROWS_LO, ROWS_HI)])

    return k(src, dst, table, zeros)


def _degrees(src, dst, ones, zeros16):
    @functools.partial(
        pl.kernel,
        out_type=(jax.ShapeDtypeStruct((NC, N, 16), jnp.float32),
                  jax.ShapeDtypeStruct((NC, N, 16), jnp.float32)),
        mesh=_mesh,
        compiler_params=pltpu.CompilerParams(use_tc_tiling_on_sc=False),
        scratch_types=[
            pltpu.VMEM_SHARED((N, 16), jnp.float32),
            pltpu.VMEM_SHARED((N, 16), jnp.float32),
            pltpu.VMEM((CHUNK,), jnp.int32),
            pltpu.VMEM((CHUNK,), jnp.int32),
            pltpu.VMEM((CHUNK, 16), jnp.float32),
        ],
    )
    def k(src_hbm, dst_hbm, ones_hbm, zeros_hbm, outs_hbm, outd_hbm,
          acc_s, acc_d, sidx, didx, ones_v):
        cid = lax.axis_index("c")
        sid = lax.axis_index("s")
        base = sid * ROWS_LO

        pltpu.sync_copy(ones_hbm, ones_v)

        @pl.when(sid < NS - 1)
        def _():
            pltpu.sync_copy(zeros_hbm.at[pl.ds(0, ROWS_LO)], acc_s.at[pl.ds(base, ROWS_LO)])
            pltpu.sync_copy(zeros_hbm.at[pl.ds(0, ROWS_LO)], acc_d.at[pl.ds(base, ROWS_LO)])

        @pl.when(sid == NS - 1)
        def _():
            pltpu.sync_copy(zeros_hbm.at[pl.ds(0, ROWS_HI)],
                            acc_s.at[pl.ds(15 * ROWS_LO, ROWS_HI)])
            pltpu.sync_copy(zeros_hbm.at[pl.ds(0, ROWS_HI)],
                            acc_d.at[pl.ds(15 * ROWS_LO, ROWS_HI)])

        plsc.subcore_barrier()

        e0 = (cid * NS + sid) * PER_TEC

        def body(i, carry):
            off = e0 + i * CHUNK
            pltpu.sync_copy(src_hbm.at[pl.ds(off, CHUNK)], sidx)
            pltpu.sync_copy(dst_hbm.at[pl.ds(off, CHUNK)], didx)
            pltpu.sync_copy(ones_v, acc_s.at[sidx], add=True)
            pltpu.sync_copy(ones_v, acc_d.at[didx], add=True)
            return carry

        lax.fori_loop(0, STEPS, body, 0)
        plsc.subcore_barrier()

        @pl.when(sid < NS - 1)
        def _():
            pltpu.sync_copy(acc_s.at[pl.ds(base, ROWS_LO)],
                            outs_hbm.at[cid, pl.ds(base, ROWS_LO)])
            pltpu.sync_copy(acc_d.at[pl.ds(base, ROWS_LO)],
                            outd_hbm.at[cid, pl.ds(base, ROWS_LO)])

        @pl.when(sid == NS - 1)
        def _():
            pltpu.sync_copy(acc_s.at[pl.ds(15 * ROWS_LO, ROWS_HI)],
                            outs_hbm.at[cid, pl.ds(15 * ROWS_LO, ROWS_HI)])
            pltpu.sync_copy(acc_d.at[pl.ds(15 * ROWS_LO, ROWS_HI)],
                            outd_hbm.at[cid, pl.ds(15 * ROWS_LO, ROWS_HI)])

    return k(src, dst, ones, zeros16)


def _tc_norms_g0(dpo, dpi, x, w0):
    def body(dpo_ref, dpi_ref, x_ref, w_ref, ns_ref, nd_ref, g0_ref):
        deg_o = (dpo_ref[0] + dpo_ref[1])[:, 0:1]
        deg_i = (dpi_ref[0] + dpi_ref[1])[:, 0:1]
        ns = lax.rsqrt(jnp.maximum(deg_o, 1.0))
        nd = lax.rsqrt(jnp.maximum(deg_i, 1.0))
        ns_ref[...] = ns
        nd_ref[...] = nd
        y = jnp.dot(x_ref[...], w_ref[...], preferred_element_type=jnp.float32)
        g0_ref[...] = y * ns

    return pl.pallas_call(
        body,
        out_shape=(jax.ShapeDtypeStruct((N, 1), jnp.float32),
                   jax.ShapeDtypeStruct((N, 1), jnp.float32),
                   jax.ShapeDtypeStruct((N, HID), jnp.float32)),
    )(dpo, dpi, x, w0)


def _tc_layer(aggp, nd, ns, b, w_next):
    def body(p_ref, nd_ref, ns_ref, b_ref, w_ref, h_ref, g_ref):
        h = (p_ref[0] + p_ref[1]) * nd_ref[...] + b_ref[...]
        h_ref[...] = h
        g_ref[...] = jnp.dot(h, w_ref[...], preferred_element_type=jnp.float32) * ns_ref[...]

    return pl.pallas_call(
        body,
        out_shape=(jax.ShapeDtypeStruct((N, HID), jnp.float32),
                   jax.ShapeDtypeStruct((N, HID), jnp.float32)),
    )(aggp, nd, ns, b, w_next)


def _tc_score_prep(aggp, nd, ns, b2, h1, h2, ws):
    def body(p_ref, nd_ref, ns_ref, b_ref, h1_ref, h2_ref, ws_ref, h3_ref, t_ref):
        h3 = (p_ref[0] + p_ref[1]) * nd_ref[...] + b_ref[...]
        h3_ref[...] = h3
        t = (jnp.dot(h1_ref[...], ws_ref[0:HID], preferred_element_type=jnp.float32)
             + jnp.dot(h2_ref[...], ws_ref[HID:2 * HID], preferred_element_type=jnp.float32)
             + jnp.dot(h3, ws_ref[2 * HID:3 * HID], preferred_element_type=jnp.float32))
        t_ref[...] = jnp.broadcast_to(t * ns_ref[...], (N, 16))

    return pl.pallas_call(
        body,
        out_shape=(jax.ShapeDtypeStruct((N, HID), jnp.float32),
                   jax.ShapeDtypeStruct((N, 16), jnp.float32)),
    )(aggp, nd, ns, b2, h1, h2, ws)


def _tc_final(sp, nd, bs, h1, h2, h3, seq, wl1, bl1, wl2, bl2, wl3, bl3):
    def body(sp_ref, nd_ref, bs_ref, h1_ref, h2_ref, h3_ref, seq_ref,
             wl1_ref, bl1_ref, wl2_ref, bl2_ref, wl3_ref, bl3_ref, out_ref):
        s16 = (sp_ref[0] + sp_ref[1]) * nd_ref[...] + bs_ref[...]
        bits = lax.bitcast_convert_type(s16, jnp.int32)
        key = bits ^ (jnp.int32(0x7FFFFFFF) & (bits >> 31))
        min_i32 = jnp.int32(-2147483648)

        def bis_a(i, acc):
            cand = acc | (jnp.int32(1) << (31 - i))
            cnt = jnp.sum((key >= (cand ^ min_i32)).astype(jnp.int32)) // 16
            return jnp.where(cnt >= K, cand, acc)

        tau_u = lax.fori_loop(0, 32, bis_a, jnp.int32(0))
        tau_s = tau_u ^ min_i32

        gt = key > tau_s
        eq = key == tau_s
        cnt_gt = jnp.sum(gt.astype(jnp.int32)) // 16
        r = K - cnt_gt

        idx = lax.broadcasted_iota(jnp.int32, (N, 16), 0)

        def bis_b(i, acc):
            cand = acc | (jnp.int32(1) << (13 - i))
            f = jnp.sum((eq & (idx < cand)).astype(jnp.int32)) // 16
            return jnp.where(f <= r, cand, acc)

        m = lax.fori_loop(0, 14, bis_b, jnp.int32(0))
        sel = gt | (eq & (idx < m))

        w16 = jnp.tanh(s16) * sel.astype(jnp.float32)
        w1 = w16[:, 0:1]
        sel1 = sel[:, 0:1]

        cat = jnp.concatenate([h1_ref[...], h2_ref[...], h3_ref[...]], axis=1)
        pooled = cat * w1
        avg = jnp.sum(pooled, axis=0, keepdims=True) * (1.0 / K)
        neg = jnp.float32(-jnp.inf)
        mx = jnp.max(jnp.where(sel1, pooled, neg), axis=0, keepdims=True)

        feat = jnp.concatenate([avg, mx, seq_ref[...]], axis=1)
        a1 = jnp.maximum(
            jnp.dot(feat, wl1_ref[...], preferred_element_type=jnp.float32)
            + bl1_ref[...], 0.0)
        a2 = jnp.maximum(
            jnp.dot(a1, wl2_ref[...], preferred_element_type=jnp.float32)
            + bl2_ref[...], 0.0)
        out_ref[...] = (jnp.dot(a2, wl3_ref[...], preferred_element_type=jnp.float32)
                        + bl3_ref[...])

    return pl.pallas_call(
        body,
        out_shape=jax.ShapeDtypeStruct((1, 128), jnp.float32),
    )(sp, nd, bs, h1, h2, h3, seq, wl1, bl1, wl2, bl2, wl3, bl3)


def kernel(x, edge_index, sequence_feature, W0, b0, W1, b1, W2, b2, Ws, bs,
           Wl1, bl1, Wl2, bl2, Wl3, bl3):
    src = edge_index[0]
    dst = edge_index[1]

    zeros128 = jnp.zeros((ZROWS, HID), jnp.float32)
    zeros16 = jnp.zeros((ZROWS, 16), jnp.float32)
    ones16 = jnp.ones((CHUNK, 16), jnp.float32)

    dpo, dpi = _degrees(src, dst, ones16, zeros16)

    ns, nd, g0 = _tc_norms_g0(dpo, dpi, x, W0)

    p0 = _mp_pass(src, dst, g0, zeros128)
    h1, g1 = _tc_layer(p0, nd, ns, b0.reshape(1, HID), W1)
    p1 = _mp_pass(src, dst, g1, zeros128)
    h2, g2 = _tc_layer(p1, nd, ns, b1.reshape(1, HID), W2)
    p2 = _mp_pass(src, dst, g2, zeros128)
    h3, t16 = _tc_score_prep(p2, nd, ns, b2.reshape(1, HID), h1, h2, Ws)

    sp = _mp_pass(src, dst, t16, zeros16)

    return _tc_final(sp, nd, bs.reshape(1, 1), h1, h2, h3, sequence_feature,
                     Wl1, bl1, Wl2, bl2, Wl3, bl3)


# speedup vs baseline: 12.5995x; 2.3311x over previous
import functools

import jax
import jax.numpy as jnp
from jax import lax
from jax.experimental import pallas as pl
from jax.experimental.pallas import tpu as pltpu
from jax.experimental.pallas import tpu_sc as plsc

N = 10000
E = 320000
HID = 128
K = 5000

NC = 2
NS = 16
NW = NC * NS
PER_TEC = E // NW
CHUNK = 80
STEPS = PER_TEC // CHUNK

ROWS_LO = 624
ROWS_HI = N - 15 * ROWS_LO
ZROWS = 640

_mesh = plsc.VectorSubcoreMesh(core_axis_name="c", subcore_axis_name="s")


def _mp_pass(src3, dst3, table, zeros):
    d = table.shape[1]

    @functools.partial(
        pl.kernel,
        out_type=jax.ShapeDtypeStruct((NC, N, d), jnp.float32),
        mesh=_mesh,
        compiler_params=pltpu.CompilerParams(use_tc_tiling_on_sc=(d == HID)),
        scratch_types=[
            pltpu.VMEM_SHARED((N, d), jnp.float32),
            pltpu.VMEM((3, CHUNK), jnp.int32),
            pltpu.VMEM((3, CHUNK), jnp.int32),
            pltpu.VMEM((2, CHUNK, d), jnp.float32),
            pltpu.SemaphoreType.DMA((2,)),
            pltpu.SemaphoreType.DMA((2,)),
            pltpu.SemaphoreType.DMA((3,)),
            pltpu.SemaphoreType.DMA((3,)),
        ],
    )
    def k(src_hbm, dst_hbm, table_hbm, zeros_hbm, out_hbm,
          acc, sidx, didx, rows, sem_g, sem_s, sem_si, sem_di):
        cid = lax.axis_index("c")
        sid = lax.axis_index("s")
        wid = cid * NS + sid
        base = sid * ROWS_LO

        @pl.when(sid < NS - 1)
        def _():
            pltpu.sync_copy(zeros_hbm.at[pl.ds(0, ROWS_LO)], acc.at[pl.ds(base, ROWS_LO)])

        @pl.when(sid == NS - 1)
        def _():
            pltpu.sync_copy(zeros_hbm.at[pl.ds(0, ROWS_HI)],
                            acc.at[pl.ds(15 * ROWS_LO, ROWS_HI)])

        plsc.subcore_barrier()

        pltpu.async_copy(src_hbm.at[wid, 0], sidx.at[0], sem_si.at[0])
        pltpu.async_copy(dst_hbm.at[wid, 0], didx.at[0], sem_di.at[0])
        pltpu.async_copy(src_hbm.at[wid, 1], sidx.at[1], sem_si.at[1])
        pltpu.async_copy(dst_hbm.at[wid, 1], didx.at[1], sem_di.at[1])
        pltpu.make_async_copy(src_hbm.at[wid, 0], sidx.at[0], sem_si.at[0]).wait()
        pltpu.async_copy(table_hbm.at[sidx.at[0]], rows.at[0], sem_g.at[0])

        def body(i, carry):
            ib = lax.rem(i, 2)
            nb = 1 - ib
            s_cur = lax.rem(i, 3)
            s_nxt = lax.rem(i + 1, 3)
            s_pre = lax.rem(i + 2, 3)

            @pl.when(i >= 1)
            def _():
                pltpu.make_async_copy(rows.at[nb], acc.at[didx.at[lax.rem(i - 1, 3)]],
                                      sem_s.at[nb]).wait()

            @pl.when(i + 2 < STEPS)
            def _():
                pltpu.async_copy(src_hbm.at[wid, i + 2], sidx.at[s_pre],
                                 sem_si.at[s_pre])
                pltpu.async_copy(dst_hbm.at[wid, i + 2], didx.at[s_pre],
                                 sem_di.at[s_pre])

            @pl.when(i + 1 < STEPS)
            def _():
                pltpu.make_async_copy(src_hbm.at[wid, i + 1], sidx.at[s_nxt],
                                      sem_si.at[s_nxt]).wait()
                pltpu.async_copy(table_hbm.at[sidx.at[s_nxt]], rows.at[nb],
                                 sem_g.at[nb])

            pltpu.make_async_copy(table_hbm.at[sidx.at[s_cur]], rows.at[ib],
                                  sem_g.at[ib]).wait()
            pltpu.make_async_copy(dst_hbm.at[wid, i], didx.at[s_cur],
                                  sem_di.at[s_cur]).wait()
            pltpu.async_copy(rows.at[ib], acc.at[didx.at[s_cur]], sem_s.at[ib],
                             add=True)
            return carry

        lax.fori_loop(0, STEPS, body, 0)
        last = lax.rem(STEPS - 1, 2)
        pltpu.make_async_copy(rows.at[last], acc.at[didx.at[lax.rem(STEPS - 1, 3)]],
                              sem_s.at[last]).wait()
        plsc.subcore_barrier()

        @pl.when(sid < NS - 1)
        def _():
            pltpu.sync_copy(acc.at[pl.ds(base, ROWS_LO)],
                            out_hbm.at[cid, pl.ds(base, ROWS_LO)])

        @pl.when(sid == NS - 1)
        def _():
            pltpu.sync_copy(acc.at[pl.ds(15 * ROWS_LO, ROWS_HI)],
                            out_hbm.at[cid, pl.ds(15 * ROWS_LO, ROWS_HI)])

    return k(src3, dst3, table, zeros)


def _degrees(src3, dst3, ones, zeros16):
    @functools.partial(
        pl.kernel,
        out_type=(jax.ShapeDtypeStruct((NC, N, 16), jnp.float32),
                  jax.ShapeDtypeStruct((NC, N, 16), jnp.float32)),
        mesh=_mesh,
        compiler_params=pltpu.CompilerParams(use_tc_tiling_on_sc=False),
        scratch_types=[
            pltpu.VMEM_SHARED((N, 16), jnp.float32),
            pltpu.VMEM_SHARED((N, 16), jnp.float32),
            pltpu.VMEM((STEPS, CHUNK), jnp.int32),
            pltpu.VMEM((STEPS, CHUNK), jnp.int32),
            pltpu.VMEM((CHUNK, 16), jnp.float32),
            pltpu.SemaphoreType.DMA((2,)),
            pltpu.SemaphoreType.DMA((2,)),
        ],
    )
    def k(src_hbm, dst_hbm, ones_hbm, zeros_hbm, outs_hbm, outd_hbm,
          acc_s, acc_d, sidx, didx, ones_v, sem_s, sem_d):
        cid = lax.axis_index("c")
        sid = lax.axis_index("s")
        wid = cid * NS + sid
        base = sid * ROWS_LO

        pltpu.sync_copy(ones_hbm, ones_v)

        @pl.when(sid < NS - 1)
        def _():
            pltpu.sync_copy(zeros_hbm.at[pl.ds(0, ROWS_LO)], acc_s.at[pl.ds(base, ROWS_LO)])
            pltpu.sync_copy(zeros_hbm.at[pl.ds(0, ROWS_LO)], acc_d.at[pl.ds(base, ROWS_LO)])

        @pl.when(sid == NS - 1)
        def _():
            pltpu.sync_copy(zeros_hbm.at[pl.ds(0, ROWS_HI)],
                            acc_s.at[pl.ds(15 * ROWS_LO, ROWS_HI)])
            pltpu.sync_copy(zeros_hbm.at[pl.ds(0, ROWS_HI)],
                            acc_d.at[pl.ds(15 * ROWS_LO, ROWS_HI)])

        pltpu.sync_copy(src_hbm.at[wid], sidx)
        pltpu.sync_copy(dst_hbm.at[wid], didx)
        plsc.subcore_barrier()

        def body(i, carry):
            ib = lax.rem(i, 2)
            nb = 1 - ib

            @pl.when(i >= 1)
            def _():
                pltpu.make_async_copy(ones_v, acc_s.at[sidx.at[i - 1]],
                                      sem_s.at[nb]).wait()
                pltpu.make_async_copy(ones_v, acc_d.at[didx.at[i - 1]],
                                      sem_d.at[nb]).wait()

            pltpu.async_copy(ones_v, acc_s.at[sidx.at[i]], sem_s.at[ib], add=True)
            pltpu.async_copy(ones_v, acc_d.at[didx.at[i]], sem_d.at[ib], add=True)
            return carry

        lax.fori_loop(0, STEPS, body, 0)
        last = lax.rem(STEPS - 1, 2)
        pltpu.make_async_copy(ones_v, acc_s.at[sidx.at[STEPS - 1]],
                              sem_s.at[last]).wait()
        pltpu.make_async_copy(ones_v, acc_d.at[didx.at[STEPS - 1]],
                              sem_d.at[last]).wait()
        plsc.subcore_barrier()

        @pl.when(sid < NS - 1)
        def _():
            pltpu.sync_copy(acc_s.at[pl.ds(base, ROWS_LO)],
                            outs_hbm.at[cid, pl.ds(base, ROWS_LO)])
            pltpu.sync_copy(acc_d.at[pl.ds(base, ROWS_LO)],
                            outd_hbm.at[cid, pl.ds(base, ROWS_LO)])

        @pl.when(sid == NS - 1)
        def _():
            pltpu.sync_copy(acc_s.at[pl.ds(15 * ROWS_LO, ROWS_HI)],
                            outs_hbm.at[cid, pl.ds(15 * ROWS_LO, ROWS_HI)])
            pltpu.sync_copy(acc_d.at[pl.ds(15 * ROWS_LO, ROWS_HI)],
                            outd_hbm.at[cid, pl.ds(15 * ROWS_LO, ROWS_HI)])

    return k(src3, dst3, ones, zeros16)


def _tc_norms_g0(dpo, dpi, x, w0):
    def body(dpo_ref, dpi_ref, x_ref, w_ref, ns_ref, nd_ref, g0_ref):
        deg_o = (dpo_ref[0] + dpo_ref[1])[:, 0:1]
        deg_i = (dpi_ref[0] + dpi_ref[1])[:, 0:1]
        ns = lax.rsqrt(jnp.maximum(deg_o, 1.0))
        nd = lax.rsqrt(jnp.maximum(deg_i, 1.0))
        ns_ref[...] = ns
        nd_ref[...] = nd
        y = jnp.dot(x_ref[...], w_ref[...], preferred_element_type=jnp.float32)
        g0_ref[...] = y * ns

    return pl.pallas_call(
        body,
        out_shape=(jax.ShapeDtypeStruct((N, 1), jnp.float32),
                   jax.ShapeDtypeStruct((N, 1), jnp.float32),
                   jax.ShapeDtypeStruct((N, HID), jnp.float32)),
    )(dpo, dpi, x, w0)


def _tc_layer(aggp, nd, ns, b, w_next):
    def body(p_ref, nd_ref, ns_ref, b_ref, w_ref, h_ref, g_ref):
        h = (p_ref[0] + p_ref[1]) * nd_ref[...] + b_ref[...]
        h_ref[...] = h
        g_ref[...] = jnp.dot(h, w_ref[...], preferred_element_type=jnp.float32) * ns_ref[...]

    return pl.pallas_call(
        body,
        out_shape=(jax.ShapeDtypeStruct((N, HID), jnp.float32),
                   jax.ShapeDtypeStruct((N, HID), jnp.float32)),
    )(aggp, nd, ns, b, w_next)


def _tc_score_prep(aggp, nd, ns, b2, h1, h2, ws):
    def body(p_ref, nd_ref, ns_ref, b_ref, h1_ref, h2_ref, ws_ref, h3_ref, t_ref):
        h3 = (p_ref[0] + p_ref[1]) * nd_ref[...] + b_ref[...]
        h3_ref[...] = h3
        t = (jnp.dot(h1_ref[...], ws_ref[0:HID], preferred_element_type=jnp.float32)
             + jnp.dot(h2_ref[...], ws_ref[HID:2 * HID], preferred_element_type=jnp.float32)
             + jnp.dot(h3, ws_ref[2 * HID:3 * HID], preferred_element_type=jnp.float32))
        t_ref[...] = jnp.broadcast_to(t * ns_ref[...], (N, 16))

    return pl.pallas_call(
        body,
        out_shape=(jax.ShapeDtypeStruct((N, HID), jnp.float32),
                   jax.ShapeDtypeStruct((N, 16), jnp.float32)),
    )(aggp, nd, ns, b2, h1, h2, ws)


def _tc_final(sp, nd, bs, h1, h2, h3, seq, wl1, bl1, wl2, bl2, wl3, bl3):
    def body(sp_ref, nd_ref, bs_ref, h1_ref, h2_ref, h3_ref, seq_ref,
             wl1_ref, bl1_ref, wl2_ref, bl2_ref, wl3_ref, bl3_ref, out_ref):
        s16 = (sp_ref[0] + sp_ref[1]) * nd_ref[...] + bs_ref[...]
        bits = lax.bitcast_convert_type(s16, jnp.int32)
        key = bits ^ (jnp.int32(0x7FFFFFFF) & (bits >> 31))
        min_i32 = jnp.int32(-2147483648)

        def bis_a(i, acc):
            cand = acc | (jnp.int32(1) << (31 - i))
            cnt = jnp.sum((key >= (cand ^ min_i32)).astype(jnp.int32)) // 16
            return jnp.where(cnt >= K, cand, acc)

        tau_u = lax.fori_loop(0, 32, bis_a, jnp.int32(0))
        tau_s = tau_u ^ min_i32

        gt = key > tau_s
        eq = key == tau_s
        cnt_gt = jnp.sum(gt.astype(jnp.int32)) // 16
        r = K - cnt_gt

        idx = lax.broadcasted_iota(jnp.int32, (N, 16), 0)

        def bis_b(i, acc):
            cand = acc | (jnp.int32(1) << (13 - i))
            f = jnp.sum((eq & (idx < cand)).astype(jnp.int32)) // 16
            return jnp.where(f <= r, cand, acc)

        m = lax.fori_loop(0, 14, bis_b, jnp.int32(0))
        sel = gt | (eq & (idx < m))

        w16 = jnp.tanh(s16) * sel.astype(jnp.float32)
        w1 = w16[:, 0:1]
        sel1 = sel[:, 0:1]

        cat = jnp.concatenate([h1_ref[...], h2_ref[...], h3_ref[...]], axis=1)
        pooled = cat * w1
        avg = jnp.sum(pooled, axis=0, keepdims=True) * (1.0 / K)
        neg = jnp.float32(-jnp.inf)
        mx = jnp.max(jnp.where(sel1, pooled, neg), axis=0, keepdims=True)

        feat = jnp.concatenate([avg, mx, seq_ref[...]], axis=1)
        a1 = jnp.maximum(
            jnp.dot(feat, wl1_ref[...], preferred_element_type=jnp.float32)
            + bl1_ref[...], 0.0)
        a2 = jnp.maximum(
            jnp.dot(a1, wl2_ref[...], preferred_element_type=jnp.float32)
            + bl2_ref[...], 0.0)
        out_ref[...] = (jnp.dot(a2, wl3_ref[...], preferred_element_type=jnp.float32)
                        + bl3_ref[...])

    return pl.pallas_call(
        body,
        out_shape=jax.ShapeDtypeStruct((1, 128), jnp.float32),
    )(sp, nd, bs, h1, h2, h3, seq, wl1, bl1, wl2, bl2, wl3, bl3)


def kernel(x, edge_index, sequence_feature, W0, b0, W1, b1, W2, b2, Ws, bs,
           Wl1, bl1, Wl2, bl2, Wl3, bl3):
    src = edge_index[0].reshape(NW, STEPS, CHUNK)
    dst = edge_index[1].reshape(NW, STEPS, CHUNK)

    zeros128 = jnp.zeros((ZROWS, HID), jnp.float32)
    zeros16 = jnp.zeros((ZROWS, 16), jnp.float32)
    ones16 = jnp.ones((CHUNK, 16), jnp.float32)

    dpo, dpi = _degrees(src, dst, ones16, zeros16)

    ns, nd, g0 = _tc_norms_g0(dpo, dpi, x, W0)

    p0 = _mp_pass(src, dst, g0, zeros128)
    h1, g1 = _tc_layer(p0, nd, ns, b0.reshape(1, HID), W1)
    p1 = _mp_pass(src, dst, g1, zeros128)
    h2, g2 = _tc_layer(p1, nd, ns, b1.reshape(1, HID), W2)
    p2 = _mp_pass(src, dst, g2, zeros128)
    h3, t16 = _tc_score_prep(p2, nd, ns, b2.reshape(1, HID), h1, h2, Ws)

    sp = _mp_pass(src, dst, t16, zeros16)

    return _tc_final(sp, nd, bs.reshape(1, 1), h1, h2, h3, sequence_feature,
                     Wl1, bl1, Wl2, bl2, Wl3, bl3)


# speedup vs baseline: 13.6492x; 1.0833x over previous
import functools

import jax
import jax.numpy as jnp
from jax import lax
from jax.experimental import pallas as pl
from jax.experimental.pallas import tpu as pltpu
from jax.experimental.pallas import tpu_sc as plsc

N = 10000
E = 320000
HID = 128
K = 5000

NC = 2
NS = 16
NW = NC * NS
CHUNK = 128
DUMMY = 512
NPAD = N + DUMMY
EPAD = 327680
PER_TEC = EPAD // NW
STEPS = PER_TEC // CHUNK

ROWS_LO = 624
ROWS_HI = N - 15 * ROWS_LO
Z_LO = 656
Z_HI = NPAD - 15 * Z_LO
ZROWS = 672

_mesh = plsc.VectorSubcoreMesh(core_axis_name="c", subcore_axis_name="s")


def _mp_pass(src3, dst3, table, zeros):
    d = table.shape[1]

    @functools.partial(
        pl.kernel,
        out_type=jax.ShapeDtypeStruct((NC, N, d), jnp.float32),
        mesh=_mesh,
        compiler_params=pltpu.CompilerParams(use_tc_tiling_on_sc=(d == HID)),
        scratch_types=[
            pltpu.VMEM_SHARED((NPAD, d), jnp.float32),
            pltpu.VMEM((3, CHUNK), jnp.int32),
            pltpu.VMEM((3, CHUNK), jnp.int32),
            pltpu.VMEM((2, CHUNK, d), jnp.float32),
            pltpu.SemaphoreType.DMA((2,)),
            pltpu.SemaphoreType.DMA((2,)),
            pltpu.SemaphoreType.DMA((3,)),
            pltpu.SemaphoreType.DMA((3,)),
        ],
    )
    def k(src_hbm, dst_hbm, table_hbm, zeros_hbm, out_hbm,
          acc, sidx, didx, rows, sem_g, sem_s, sem_si, sem_di):
        cid = lax.axis_index("c")
        sid = lax.axis_index("s")
        wid = cid * NS + sid
        base = sid * ROWS_LO

        @pl.when(sid < NS - 1)
        def _():
            pltpu.sync_copy(zeros_hbm.at[pl.ds(0, Z_LO)],
                            acc.at[pl.ds(sid * Z_LO, Z_LO)])

        @pl.when(sid == NS - 1)
        def _():
            pltpu.sync_copy(zeros_hbm.at[pl.ds(0, Z_HI)],
                            acc.at[pl.ds(15 * Z_LO, Z_HI)])

        plsc.subcore_barrier()

        pltpu.async_copy(src_hbm.at[wid, 0], sidx.at[0], sem_si.at[0])
        pltpu.async_copy(dst_hbm.at[wid, 0], didx.at[0], sem_di.at[0])
        pltpu.async_copy(src_hbm.at[wid, 1], sidx.at[1], sem_si.at[1])
        pltpu.async_copy(dst_hbm.at[wid, 1], didx.at[1], sem_di.at[1])
        pltpu.make_async_copy(src_hbm.at[wid, 0], sidx.at[0], sem_si.at[0]).wait()
        pltpu.async_copy(table_hbm.at[sidx.at[0]], rows.at[0], sem_g.at[0])

        def body(i, carry):
            ib = lax.rem(i, 2)
            nb = 1 - ib
            s_cur = lax.rem(i, 3)
            s_nxt = lax.rem(i + 1, 3)
            s_pre = lax.rem(i + 2, 3)

            @pl.when(i >= 1)
            def _():
                pltpu.make_async_copy(rows.at[nb], acc.at[didx.at[lax.rem(i - 1, 3)]],
                                      sem_s.at[nb]).wait()

            @pl.when(i + 2 < STEPS)
            def _():
                pltpu.async_copy(src_hbm.at[wid, i + 2], sidx.at[s_pre],
                                 sem_si.at[s_pre])
                pltpu.async_copy(dst_hbm.at[wid, i + 2], didx.at[s_pre],
                                 sem_di.at[s_pre])

            @pl.when(i + 1 < STEPS)
            def _():
                pltpu.make_async_copy(src_hbm.at[wid, i + 1], sidx.at[s_nxt],
                                      sem_si.at[s_nxt]).wait()
                pltpu.async_copy(table_hbm.at[sidx.at[s_nxt]], rows.at[nb],
                                 sem_g.at[nb])

            pltpu.make_async_copy(table_hbm.at[sidx.at[s_cur]], rows.at[ib],
                                  sem_g.at[ib]).wait()
            pltpu.make_async_copy(dst_hbm.at[wid, i], didx.at[s_cur],
                                  sem_di.at[s_cur]).wait()
            pltpu.async_copy(rows.at[ib], acc.at[didx.at[s_cur]], sem_s.at[ib],
                             add=True)
            return carry

        lax.fori_loop(0, STEPS, body, 0)
        last = lax.rem(STEPS - 1, 2)
        pltpu.make_async_copy(rows.at[last], acc.at[didx.at[lax.rem(STEPS - 1, 3)]],
                              sem_s.at[last]).wait()
        plsc.subcore_barrier()

        @pl.when(sid < NS - 1)
        def _():
            pltpu.sync_copy(acc.at[pl.ds(base, ROWS_LO)],
                            out_hbm.at[cid, pl.ds(base, ROWS_LO)])

        @pl.when(sid == NS - 1)
        def _():
            pltpu.sync_copy(acc.at[pl.ds(15 * ROWS_LO, ROWS_HI)],
                            out_hbm.at[cid, pl.ds(15 * ROWS_LO, ROWS_HI)])

    return k(src3, dst3, table, zeros)


def _degrees(src3, dst3, ones, zeros16):
    @functools.partial(
        pl.kernel,
        out_type=(jax.ShapeDtypeStruct((NC, N, 16), jnp.float32),
                  jax.ShapeDtypeStruct((NC, N, 16), jnp.float32)),
        mesh=_mesh,
        compiler_params=pltpu.CompilerParams(use_tc_tiling_on_sc=False),
        scratch_types=[
            pltpu.VMEM_SHARED((NPAD, 16), jnp.float32),
            pltpu.VMEM_SHARED((NPAD, 16), jnp.float32),
            pltpu.VMEM((STEPS, CHUNK), jnp.int32),
            pltpu.VMEM((STEPS, CHUNK), jnp.int32),
            pltpu.VMEM((CHUNK, 16), jnp.float32),
            pltpu.SemaphoreType.DMA((2,)),
            pltpu.SemaphoreType.DMA((2,)),
        ],
    )
    def k(src_hbm, dst_hbm, ones_hbm, zeros_hbm, outs_hbm, outd_hbm,
          acc_s, acc_d, sidx, didx, ones_v, sem_s, sem_d):
        cid = lax.axis_index("c")
        sid = lax.axis_index("s")
        wid = cid * NS + sid
        base = sid * ROWS_LO

        pltpu.sync_copy(ones_hbm, ones_v)

        @pl.when(sid < NS - 1)
        def _():
            pltpu.sync_copy(zeros_hbm.at[pl.ds(0, Z_LO)],
                            acc_s.at[pl.ds(sid * Z_LO, Z_LO)])
            pltpu.sync_copy(zeros_hbm.at[pl.ds(0, Z_LO)],
                            acc_d.at[pl.ds(sid * Z_LO, Z_LO)])

        @pl.when(sid == NS - 1)
        def _():
            pltpu.sync_copy(zeros_hbm.at[pl.ds(0, Z_HI)],
                            acc_s.at[pl.ds(15 * Z_LO, Z_HI)])
            pltpu.sync_copy(zeros_hbm.at[pl.ds(0, Z_HI)],
                            acc_d.at[pl.ds(15 * Z_LO, Z_HI)])

        pltpu.sync_copy(src_hbm.at[wid], sidx)
        pltpu.sync_copy(dst_hbm.at[wid], didx)
        plsc.subcore_barrier()

        def body(i, carry):
            ib = lax.rem(i, 2)
            nb = 1 - ib

            @pl.when(i >= 1)
            def _():
                pltpu.make_async_copy(ones_v, acc_s.at[sidx.at[i - 1]],
                                      sem_s.at[nb]).wait()
                pltpu.make_async_copy(ones_v, acc_d.at[didx.at[i - 1]],
                                      sem_d.at[nb]).wait()

            pltpu.async_copy(ones_v, acc_s.at[sidx.at[i]], sem_s.at[ib], add=True)
            pltpu.async_copy(ones_v, acc_d.at[didx.at[i]], sem_d.at[ib], add=True)
            return carry

        lax.fori_loop(0, STEPS, body, 0)
        last = lax.rem(STEPS - 1, 2)
        pltpu.make_async_copy(ones_v, acc_s.at[sidx.at[STEPS - 1]],
                              sem_s.at[last]).wait()
        pltpu.make_async_copy(ones_v, acc_d.at[didx.at[STEPS - 1]],
                              sem_d.at[last]).wait()
        plsc.subcore_barrier()

        @pl.when(sid < NS - 1)
        def _():
            pltpu.sync_copy(acc_s.at[pl.ds(base, ROWS_LO)],
                            outs_hbm.at[cid, pl.ds(base, ROWS_LO)])
            pltpu.sync_copy(acc_d.at[pl.ds(base, ROWS_LO)],
                            outd_hbm.at[cid, pl.ds(base, ROWS_LO)])

        @pl.when(sid == NS - 1)
        def _():
            pltpu.sync_copy(acc_s.at[pl.ds(15 * ROWS_LO, ROWS_HI)],
                            outs_hbm.at[cid, pl.ds(15 * ROWS_LO, ROWS_HI)])
            pltpu.sync_copy(acc_d.at[pl.ds(15 * ROWS_LO, ROWS_HI)],
                            outd_hbm.at[cid, pl.ds(15 * ROWS_LO, ROWS_HI)])

    return k(src3, dst3, ones, zeros16)


def _tc_norms_g0(dpo, dpi, x, w0):
    def body(dpo_ref, dpi_ref, x_ref, w_ref, ns_ref, nd_ref, g0_ref):
        deg_o = (dpo_ref[0] + dpo_ref[1])[:, 0:1]
        deg_i = (dpi_ref[0] + dpi_ref[1])[:, 0:1]
        ns = lax.rsqrt(jnp.maximum(deg_o, 1.0))
        nd = lax.rsqrt(jnp.maximum(deg_i, 1.0))
        ns_ref[...] = ns
        nd_ref[...] = nd
        y = jnp.dot(x_ref[...], w_ref[...], preferred_element_type=jnp.float32)
        g0_ref[0:N, :] = y * ns
        g0_ref[N:NPAD, :] = jnp.zeros((DUMMY, HID), jnp.float32)

    return pl.pallas_call(
        body,
        out_shape=(jax.ShapeDtypeStruct((N, 1), jnp.float32),
                   jax.ShapeDtypeStruct((N, 1), jnp.float32),
                   jax.ShapeDtypeStruct((NPAD, HID), jnp.float32)),
    )(dpo, dpi, x, w0)


def _tc_layer(aggp, nd, ns, b, w_next):
    def body(p_ref, nd_ref, ns_ref, b_ref, w_ref, h_ref, g_ref):
        h = (p_ref[0] + p_ref[1]) * nd_ref[...] + b_ref[...]
        h_ref[...] = h
        g_ref[0:N, :] = jnp.dot(h, w_ref[...], preferred_element_type=jnp.float32) * ns_ref[...]
        g_ref[N:NPAD, :] = jnp.zeros((DUMMY, HID), jnp.float32)

    return pl.pallas_call(
        body,
        out_shape=(jax.ShapeDtypeStruct((N, HID), jnp.float32),
                   jax.ShapeDtypeStruct((NPAD, HID), jnp.float32)),
    )(aggp, nd, ns, b, w_next)


def _tc_score_prep(aggp, nd, ns, b2, h1, h2, ws):
    def body(p_ref, nd_ref, ns_ref, b_ref, h1_ref, h2_ref, ws_ref, h3_ref, t_ref):
        h3 = (p_ref[0] + p_ref[1]) * nd_ref[...] + b_ref[...]
        h3_ref[...] = h3
        t = (jnp.dot(h1_ref[...], ws_ref[0:HID], preferred_element_type=jnp.float32)
             + jnp.dot(h2_ref[...], ws_ref[HID:2 * HID], preferred_element_type=jnp.float32)
             + jnp.dot(h3, ws_ref[2 * HID:3 * HID], preferred_element_type=jnp.float32))
        t_ref[0:N, :] = jnp.broadcast_to(t * ns_ref[...], (N, 16))
        t_ref[N:NPAD, :] = jnp.zeros((DUMMY, 16), jnp.float32)

    return pl.pallas_call(
        body,
        out_shape=(jax.ShapeDtypeStruct((N, HID), jnp.float32),
                   jax.ShapeDtypeStruct((NPAD, 16), jnp.float32)),
    )(aggp, nd, ns, b2, h1, h2, ws)


def _tc_final(sp, nd, bs, h1, h2, h3, seq, wl1, bl1, wl2, bl2, wl3, bl3):
    def body(sp_ref, nd_ref, bs_ref, h1_ref, h2_ref, h3_ref, seq_ref,
             wl1_ref, bl1_ref, wl2_ref, bl2_ref, wl3_ref, bl3_ref, out_ref):
        s16 = (sp_ref[0] + sp_ref[1]) * nd_ref[...] + bs_ref[...]
        bits = lax.bitcast_convert_type(s16, jnp.int32)
        key = bits ^ (jnp.int32(0x7FFFFFFF) & (bits >> 31))
        min_i32 = jnp.int32(-2147483648)

        def bis_a(i, acc):
            cand = acc | (jnp.int32(1) << (31 - i))
            cnt = jnp.sum((key >= (cand ^ min_i32)).astype(jnp.int32)) // 16
            return jnp.where(cnt >= K, cand, acc)

        tau_u = lax.fori_loop(0, 32, bis_a, jnp.int32(0))
        tau_s = tau_u ^ min_i32

        gt = key > tau_s
        eq = key == tau_s
        cnt_gt = jnp.sum(gt.astype(jnp.int32)) // 16
        r = K - cnt_gt

        idx = lax.broadcasted_iota(jnp.int32, (N, 16), 0)

        def bis_b(i, acc):
            cand = acc | (jnp.int32(1) << (13 - i))
            f = jnp.sum((eq & (idx < cand)).astype(jnp.int32)) // 16
            return jnp.where(f <= r, cand, acc)

        m = lax.fori_loop(0, 14, bis_b, jnp.int32(0))
        sel = gt | (eq & (idx < m))

        w16 = jnp.tanh(s16) * sel.astype(jnp.float32)
        w1 = w16[:, 0:1]
        sel1 = sel[:, 0:1]

        cat = jnp.concatenate([h1_ref[...], h2_ref[...], h3_ref[...]], axis=1)
        pooled = cat * w1
        avg = jnp.sum(pooled, axis=0, keepdims=True) * (1.0 / K)
        neg = jnp.float32(-jnp.inf)
        mx = jnp.max(jnp.where(sel1, pooled, neg), axis=0, keepdims=True)

        feat = jnp.concatenate([avg, mx, seq_ref[...]], axis=1)
        a1 = jnp.maximum(
            jnp.dot(feat, wl1_ref[...], preferred_element_type=jnp.float32)
            + bl1_ref[...], 0.0)
        a2 = jnp.maximum(
            jnp.dot(a1, wl2_ref[...], preferred_element_type=jnp.float32)
            + bl2_ref[...], 0.0)
        out_ref[...] = (jnp.dot(a2, wl3_ref[...], preferred_element_type=jnp.float32)
                        + bl3_ref[...])

    return pl.pallas_call(
        body,
        out_shape=jax.ShapeDtypeStruct((1, 128), jnp.float32),
    )(sp, nd, bs, h1, h2, h3, seq, wl1, bl1, wl2, bl2, wl3, bl3)


def kernel(x, edge_index, sequence_feature, W0, b0, W1, b1, W2, b2, Ws, bs,
           Wl1, bl1, Wl2, bl2, Wl3, bl3):
    pad = (N + (jnp.arange(EPAD - E, dtype=jnp.int32) % DUMMY)).astype(jnp.int32)
    src = jnp.concatenate([edge_index[0], pad]).reshape(NW, STEPS, CHUNK)
    dst = jnp.concatenate([edge_index[1], pad]).reshape(NW, STEPS, CHUNK)

    zeros128 = jnp.zeros((ZROWS, HID), jnp.float32)
    zeros16 = jnp.zeros((ZROWS, 16), jnp.float32)
    ones16 = jnp.ones((CHUNK, 16), jnp.float32)

    dpo, dpi = _degrees(src, dst, ones16, zeros16)

    ns, nd, g0 = _tc_norms_g0(dpo, dpi, x, W0)

    p0 = _mp_pass(src, dst, g0, zeros128)
    h1, g1 = _tc_layer(p0, nd, ns, b0.reshape(1, HID), W1)
    p1 = _mp_pass(src, dst, g1, zeros128)
    h2, g2 = _tc_layer(p1, nd, ns, b1.reshape(1, HID), W2)
    p2 = _mp_pass(src, dst, g2, zeros128)
    h3, t16 = _tc_score_prep(p2, nd, ns, b2.reshape(1, HID), h1, h2, Ws)

    sp = _mp_pass(src, dst, t16, zeros16)

    return _tc_final(sp, nd, bs.reshape(1, 1), h1, h2, h3, sequence_feature,
                     Wl1, bl1, Wl2, bl2, Wl3, bl3)


# speedup vs baseline: 13.6502x; 1.0001x over previous
import functools

import jax
import jax.numpy as jnp
from jax import lax
from jax.experimental import pallas as pl
from jax.experimental.pallas import tpu as pltpu
from jax.experimental.pallas import tpu_sc as plsc

N = 10000
E = 320000
HID = 128
K = 5000

NC = 2
NS = 16
NW = NC * NS
CHUNK = 128
DUMMY = 512
NPAD = N + DUMMY
EPAD = 327680
PER_TEC = EPAD // NW
STEPS = PER_TEC // CHUNK

ROWS_LO = 624
ROWS_HI = N - 15 * ROWS_LO
Z_LO = 656
Z_HI = NPAD - 15 * Z_LO
ZROWS = 672

_mesh = plsc.VectorSubcoreMesh(core_axis_name="c", subcore_axis_name="s")


def _mp_pass(src3, dst3, table, zeros):
    d = table.shape[1]
    staged = d != HID

    @functools.partial(
        pl.kernel,
        out_type=jax.ShapeDtypeStruct((NC, N, d), jnp.float32),
        mesh=_mesh,
        compiler_params=pltpu.CompilerParams(use_tc_tiling_on_sc=(d == HID)),
        scratch_types=[
            pltpu.VMEM_SHARED((NPAD, d), jnp.float32),
            pltpu.VMEM((STEPS, CHUNK) if staged else (3, CHUNK), jnp.int32),
            pltpu.VMEM((STEPS, CHUNK) if staged else (3, CHUNK), jnp.int32),
            pltpu.VMEM((2, CHUNK, d), jnp.float32),
            pltpu.SemaphoreType.DMA((2,)),
            pltpu.SemaphoreType.DMA((2,)),
            pltpu.SemaphoreType.DMA((3,)),
            pltpu.SemaphoreType.DMA((3,)),
        ],
    )
    def k(src_hbm, dst_hbm, table_hbm, zeros_hbm, out_hbm,
          acc, sidx, didx, rows, sem_g, sem_s, sem_si, sem_di):
        cid = lax.axis_index("c")
        sid = lax.axis_index("s")
        wid = cid * NS + sid
        base = sid * ROWS_LO

        @pl.when(sid < NS - 1)
        def _():
            pltpu.sync_copy(zeros_hbm.at[pl.ds(0, Z_LO)],
                            acc.at[pl.ds(sid * Z_LO, Z_LO)])

        @pl.when(sid == NS - 1)
        def _():
            pltpu.sync_copy(zeros_hbm.at[pl.ds(0, Z_HI)],
                            acc.at[pl.ds(15 * Z_LO, Z_HI)])

        plsc.subcore_barrier()

        if staged:
            pltpu.sync_copy(src_hbm.at[wid], sidx)
            pltpu.sync_copy(dst_hbm.at[wid], didx)
            pltpu.async_copy(table_hbm.at[sidx.at[0]], rows.at[0], sem_g.at[0])

            def body(i, carry):
                ib = lax.rem(i, 2)
                nb = 1 - ib

                @pl.when(i >= 1)
                def _():
                    pltpu.make_async_copy(rows.at[nb], acc.at[didx.at[i - 1]],
                                          sem_s.at[nb]).wait()

                @pl.when(i + 1 < STEPS)
                def _():
                    pltpu.async_copy(table_hbm.at[sidx.at[i + 1]], rows.at[nb],
                                     sem_g.at[nb])

                pltpu.make_async_copy(table_hbm.at[sidx.at[i]], rows.at[ib],
                                      sem_g.at[ib]).wait()
                pltpu.async_copy(rows.at[ib], acc.at[didx.at[i]], sem_s.at[ib],
                                 add=True)
                return carry

            lax.fori_loop(0, STEPS, body, 0)
            last = lax.rem(STEPS - 1, 2)
            pltpu.make_async_copy(rows.at[last], acc.at[didx.at[STEPS - 1]],
                                  sem_s.at[last]).wait()
        else:
            pltpu.async_copy(src_hbm.at[wid, 0], sidx.at[0], sem_si.at[0])
            pltpu.async_copy(dst_hbm.at[wid, 0], didx.at[0], sem_di.at[0])
            pltpu.async_copy(src_hbm.at[wid, 1], sidx.at[1], sem_si.at[1])
            pltpu.async_copy(dst_hbm.at[wid, 1], didx.at[1], sem_di.at[1])
            pltpu.make_async_copy(src_hbm.at[wid, 0], sidx.at[0], sem_si.at[0]).wait()
            pltpu.async_copy(table_hbm.at[sidx.at[0]], rows.at[0], sem_g.at[0])

            def body(i, carry):
                ib = lax.rem(i, 2)
                nb = 1 - ib
                s_cur = lax.rem(i, 3)
                s_nxt = lax.rem(i + 1, 3)
                s_pre = lax.rem(i + 2, 3)

                @pl.when(i >= 1)
                def _():
                    pltpu.make_async_copy(rows.at[nb], acc.at[didx.at[lax.rem(i - 1, 3)]],
                                          sem_s.at[nb]).wait()

                @pl.when(i + 2 < STEPS)
                def _():
                    pltpu.async_copy(src_hbm.at[wid, i + 2], sidx.at[s_pre],
                                     sem_si.at[s_pre])
                    pltpu.async_copy(dst_hbm.at[wid, i + 2], didx.at[s_pre],
                                     sem_di.at[s_pre])

                @pl.when(i + 1 < STEPS)
                def _():
                    pltpu.make_async_copy(src_hbm.at[wid, i + 1], sidx.at[s_nxt],
                                          sem_si.at[s_nxt]).wait()
                    pltpu.async_copy(table_hbm.at[sidx.at[s_nxt]], rows.at[nb],
                                     sem_g.at[nb])

                pltpu.make_async_copy(table_hbm.at[sidx.at[s_cur]], rows.at[ib],
                                      sem_g.at[ib]).wait()
                pltpu.make_async_copy(dst_hbm.at[wid, i], didx.at[s_cur],
                                      sem_di.at[s_cur]).wait()
                pltpu.async_copy(rows.at[ib], acc.at[didx.at[s_cur]], sem_s.at[ib],
                                 add=True)
                return carry

            lax.fori_loop(0, STEPS, body, 0)
            last = lax.rem(STEPS - 1, 2)
            pltpu.make_async_copy(rows.at[last], acc.at[didx.at[lax.rem(STEPS - 1, 3)]],
                                  sem_s.at[last]).wait()
        plsc.subcore_barrier()

        @pl.when(sid < NS - 1)
        def _():
            pltpu.sync_copy(acc.at[pl.ds(base, ROWS_LO)],
                            out_hbm.at[cid, pl.ds(base, ROWS_LO)])

        @pl.when(sid == NS - 1)
        def _():
            pltpu.sync_copy(acc.at[pl.ds(15 * ROWS_LO, ROWS_HI)],
                            out_hbm.at[cid, pl.ds(15 * ROWS_LO, ROWS_HI)])

    return k(src3, dst3, table, zeros)


def _degrees(src3, dst3, ones, zeros16):
    @functools.partial(
        pl.kernel,
        out_type=(jax.ShapeDtypeStruct((NC, N, 16), jnp.float32),
                  jax.ShapeDtypeStruct((NC, N, 16), jnp.float32)),
        mesh=_mesh,
        compiler_params=pltpu.CompilerParams(use_tc_tiling_on_sc=False),
        scratch_types=[
            pltpu.VMEM_SHARED((NPAD, 16), jnp.float32),
            pltpu.VMEM_SHARED((NPAD, 16), jnp.float32),
            pltpu.VMEM((STEPS, CHUNK), jnp.int32),
            pltpu.VMEM((STEPS, CHUNK), jnp.int32),
            pltpu.VMEM((CHUNK, 16), jnp.float32),
            pltpu.SemaphoreType.DMA((2,)),
            pltpu.SemaphoreType.DMA((2,)),
        ],
    )
    def k(src_hbm, dst_hbm, ones_hbm, zeros_hbm, outs_hbm, outd_hbm,
          acc_s, acc_d, sidx, didx, ones_v, sem_s, sem_d):
        cid = lax.axis_index("c")
        sid = lax.axis_index("s")
        wid = cid * NS + sid
        base = sid * ROWS_LO

        pltpu.sync_copy(ones_hbm, ones_v)

        @pl.when(sid < NS - 1)
        def _():
            pltpu.sync_copy(zeros_hbm.at[pl.ds(0, Z_LO)],
                            acc_s.at[pl.ds(sid * Z_LO, Z_LO)])
            pltpu.sync_copy(zeros_hbm.at[pl.ds(0, Z_LO)],
                            acc_d.at[pl.ds(sid * Z_LO, Z_LO)])

        @pl.when(sid == NS - 1)
        def _():
            pltpu.sync_copy(zeros_hbm.at[pl.ds(0, Z_HI)],
                            acc_s.at[pl.ds(15 * Z_LO, Z_HI)])
            pltpu.sync_copy(zeros_hbm.at[pl.ds(0, Z_HI)],
                            acc_d.at[pl.ds(15 * Z_LO, Z_HI)])

        pltpu.sync_copy(src_hbm.at[wid], sidx)
        pltpu.sync_copy(dst_hbm.at[wid], didx)
        plsc.subcore_barrier()

        def body(i, carry):
            ib = lax.rem(i, 2)
            nb = 1 - ib

            @pl.when(i >= 1)
            def _():
                pltpu.make_async_copy(ones_v, acc_s.at[sidx.at[i - 1]],
                                      sem_s.at[nb]).wait()
                pltpu.make_async_copy(ones_v, acc_d.at[didx.at[i - 1]],
                                      sem_d.at[nb]).wait()

            pltpu.async_copy(ones_v, acc_s.at[sidx.at[i]], sem_s.at[ib], add=True)
            pltpu.async_copy(ones_v, acc_d.at[didx.at[i]], sem_d.at[ib], add=True)
            return carry

        lax.fori_loop(0, STEPS, body, 0)
        last = lax.rem(STEPS - 1, 2)
        pltpu.make_async_copy(ones_v, acc_s.at[sidx.at[STEPS - 1]],
                              sem_s.at[last]).wait()
        pltpu.make_async_copy(ones_v, acc_d.at[didx.at[STEPS - 1]],
                              sem_d.at[last]).wait()
        plsc.subcore_barrier()

        @pl.when(sid < NS - 1)
        def _():
            pltpu.sync_copy(acc_s.at[pl.ds(base, ROWS_LO)],
                            outs_hbm.at[cid, pl.ds(base, ROWS_LO)])
            pltpu.sync_copy(acc_d.at[pl.ds(base, ROWS_LO)],
                            outd_hbm.at[cid, pl.ds(base, ROWS_LO)])

        @pl.when(sid == NS - 1)
        def _():
            pltpu.sync_copy(acc_s.at[pl.ds(15 * ROWS_LO, ROWS_HI)],
                            outs_hbm.at[cid, pl.ds(15 * ROWS_LO, ROWS_HI)])
            pltpu.sync_copy(acc_d.at[pl.ds(15 * ROWS_LO, ROWS_HI)],
                            outd_hbm.at[cid, pl.ds(15 * ROWS_LO, ROWS_HI)])

    return k(src3, dst3, ones, zeros16)


def _tc_norms_g0(dpo, dpi, x, w0):
    def body(dpo_ref, dpi_ref, x_ref, w_ref, ns_ref, nd_ref, g0_ref):
        deg_o = (dpo_ref[0] + dpo_ref[1])[:, 0:1]
        deg_i = (dpi_ref[0] + dpi_ref[1])[:, 0:1]
        ns = lax.rsqrt(jnp.maximum(deg_o, 1.0))
        nd = lax.rsqrt(jnp.maximum(deg_i, 1.0))
        ns_ref[...] = ns
        nd_ref[...] = nd
        y = jnp.dot(x_ref[...], w_ref[...], preferred_element_type=jnp.float32)
        g0_ref[0:N, :] = y * ns
        g0_ref[N:NPAD, :] = jnp.zeros((DUMMY, HID), jnp.float32)

    return pl.pallas_call(
        body,
        out_shape=(jax.ShapeDtypeStruct((N, 1), jnp.float32),
                   jax.ShapeDtypeStruct((N, 1), jnp.float32),
                   jax.ShapeDtypeStruct((NPAD, HID), jnp.float32)),
    )(dpo, dpi, x, w0)


def _tc_layer(aggp, nd, ns, b, w_next):
    def body(p_ref, nd_ref, ns_ref, b_ref, w_ref, h_ref, g_ref):
        h = (p_ref[0] + p_ref[1]) * nd_ref[...] + b_ref[...]
        h_ref[...] = h
        g_ref[0:N, :] = jnp.dot(h, w_ref[...], preferred_element_type=jnp.float32) * ns_ref[...]
        g_ref[N:NPAD, :] = jnp.zeros((DUMMY, HID), jnp.float32)

    return pl.pallas_call(
        body,
        out_shape=(jax.ShapeDtypeStruct((N, HID), jnp.float32),
                   jax.ShapeDtypeStruct((NPAD, HID), jnp.float32)),
    )(aggp, nd, ns, b, w_next)


def _tc_score_prep(aggp, nd, ns, b2, h1, h2, ws):
    def body(p_ref, nd_ref, ns_ref, b_ref, h1_ref, h2_ref, ws_ref, h3_ref, t_ref):
        h3 = (p_ref[0] + p_ref[1]) * nd_ref[...] + b_ref[...]
        h3_ref[...] = h3
        t = (jnp.dot(h1_ref[...], ws_ref[0:HID], preferred_element_type=jnp.float32)
             + jnp.dot(h2_ref[...], ws_ref[HID:2 * HID], preferred_element_type=jnp.float32)
             + jnp.dot(h3, ws_ref[2 * HID:3 * HID], preferred_element_type=jnp.float32))
        t_ref[0:N, :] = jnp.broadcast_to(t * ns_ref[...], (N, 16))
        t_ref[N:NPAD, :] = jnp.zeros((DUMMY, 16), jnp.float32)

    return pl.pallas_call(
        body,
        out_shape=(jax.ShapeDtypeStruct((N, HID), jnp.float32),
                   jax.ShapeDtypeStruct((NPAD, 16), jnp.float32)),
    )(aggp, nd, ns, b2, h1, h2, ws)


def _tc_final(sp, nd, bs, h1, h2, h3, seq, wl1, bl1, wl2, bl2, wl3, bl3):
    def body(sp_ref, nd_ref, bs_ref, h1_ref, h2_ref, h3_ref, seq_ref,
             wl1_ref, bl1_ref, wl2_ref, bl2_ref, wl3_ref, bl3_ref, out_ref):
        s16 = (sp_ref[0] + sp_ref[1]) * nd_ref[...] + bs_ref[...]
        bits = lax.bitcast_convert_type(s16, jnp.int32)
        key = bits ^ (jnp.int32(0x7FFFFFFF) & (bits >> 31))
        min_i32 = jnp.int32(-2147483648)

        def bis_a(i, acc):
            cand = acc | (jnp.int32(1) << (31 - i))
            cnt = jnp.sum((key >= (cand ^ min_i32)).astype(jnp.int32)) // 16
            return jnp.where(cnt >= K, cand, acc)

        tau_u = lax.fori_loop(0, 32, bis_a, jnp.int32(0))
        tau_s = tau_u ^ min_i32

        gt = key > tau_s
        eq = key == tau_s
        cnt_gt = jnp.sum(gt.astype(jnp.int32)) // 16
        r = K - cnt_gt

        idx = lax.broadcasted_iota(jnp.int32, (N, 16), 0)

        def bis_b(i, acc):
            cand = acc | (jnp.int32(1) << (13 - i))
            f = jnp.sum((eq & (idx < cand)).astype(jnp.int32)) // 16
            return jnp.where(f <= r, cand, acc)

        m = lax.fori_loop(0, 14, bis_b, jnp.int32(0))
        sel = gt | (eq & (idx < m))

        w16 = jnp.tanh(s16) * sel.astype(jnp.float32)
        w1 = w16[:, 0:1]
        sel1 = sel[:, 0:1]

        cat = jnp.concatenate([h1_ref[...], h2_ref[...], h3_ref[...]], axis=1)
        pooled = cat * w1
        avg = jnp.sum(pooled, axis=0, keepdims=True) * (1.0 / K)
        neg = jnp.float32(-jnp.inf)
        mx = jnp.max(jnp.where(sel1, pooled, neg), axis=0, keepdims=True)

        feat = jnp.concatenate([avg, mx, seq_ref[...]], axis=1)
        a1 = jnp.maximum(
            jnp.dot(feat, wl1_ref[...], preferred_element_type=jnp.float32)
            + bl1_ref[...], 0.0)
        a2 = jnp.maximum(
            jnp.dot(a1, wl2_ref[...], preferred_element_type=jnp.float32)
            + bl2_ref[...], 0.0)
        out_ref[...] = (jnp.dot(a2, wl3_ref[...], preferred_element_type=jnp.float32)
                        + bl3_ref[...])

    return pl.pallas_call(
        body,
        out_shape=jax.ShapeDtypeStruct((1, 128), jnp.float32),
    )(sp, nd, bs, h1, h2, h3, seq, wl1, bl1, wl2, bl2, wl3, bl3)


def kernel(x, edge_index, sequence_feature, W0, b0, W1, b1, W2, b2, Ws, bs,
           Wl1, bl1, Wl2, bl2, Wl3, bl3):
    pad = (N + (jnp.arange(EPAD - E, dtype=jnp.int32) % DUMMY)).astype(jnp.int32)
    src = jnp.concatenate([edge_index[0], pad]).reshape(NW, STEPS, CHUNK)
    dst = jnp.concatenate([edge_index[1], pad]).reshape(NW, STEPS, CHUNK)

    zeros128 = jnp.zeros((ZROWS, HID), jnp.float32)
    zeros16 = jnp.zeros((ZROWS, 16), jnp.float32)
    ones16 = jnp.ones((CHUNK, 16), jnp.float32)

    dpo, dpi = _degrees(src, dst, ones16, zeros16)

    ns, nd, g0 = _tc_norms_g0(dpo, dpi, x, W0)

    p0 = _mp_pass(src, dst, g0, zeros128)
    h1, g1 = _tc_layer(p0, nd, ns, b0.reshape(1, HID), W1)
    p1 = _mp_pass(src, dst, g1, zeros128)
    h2, g2 = _tc_layer(p1, nd, ns, b1.reshape(1, HID), W2)
    p2 = _mp_pass(src, dst, g2, zeros128)
    h3, t16 = _tc_score_prep(p2, nd, ns, b2.reshape(1, HID), h1, h2, Ws)

    sp = _mp_pass(src, dst, t16, zeros16)

    return _tc_final(sp, nd, bs.reshape(1, 1), h1, h2, h3, sequence_feature,
                     Wl1, bl1, Wl2, bl2, Wl3, bl3)


# speedup vs baseline: 14.9375x; 1.0943x over previous
import functools

import jax
import jax.numpy as jnp
from jax import lax
from jax.experimental import pallas as pl
from jax.experimental.pallas import tpu as pltpu
from jax.experimental.pallas import tpu_sc as plsc

N = 10000
E = 320000
HID = 128
K = 5000

NC = 2
NS = 16
NW = NC * NS
CHUNK = 128
DUMMY = 512
NPAD = N + DUMMY
EPAD = 327680
PER_TEC = EPAD // NW
STEPS = PER_TEC // CHUNK
SUPER = 1024
STEPS_E = PER_TEC // SUPER

ROWS_LO = 624
ROWS_HI = N - 15 * ROWS_LO
Z_LO = 656
Z_HI = NPAD - 15 * Z_LO
ZROWS = 672

_mesh = plsc.VectorSubcoreMesh(core_axis_name="c", subcore_axis_name="s")


def _mp_pass(src3, dst3, table, zeros):
    d = table.shape[1]

    @functools.partial(
        pl.kernel,
        out_type=jax.ShapeDtypeStruct((NC, N, d), jnp.float32),
        mesh=_mesh,
        scratch_types=[
            pltpu.VMEM_SHARED((NPAD, d), jnp.float32),
            pltpu.VMEM((3, CHUNK), jnp.int32),
            pltpu.VMEM((3, CHUNK), jnp.int32),
            pltpu.VMEM((2, CHUNK, d), jnp.float32),
            pltpu.SemaphoreType.DMA((2,)),
            pltpu.SemaphoreType.DMA((2,)),
            pltpu.SemaphoreType.DMA((3,)),
            pltpu.SemaphoreType.DMA((3,)),
        ],
    )
    def k(src_hbm, dst_hbm, table_hbm, zeros_hbm, out_hbm,
          acc, sidx, didx, rows, sem_g, sem_s, sem_si, sem_di):
        cid = lax.axis_index("c")
        sid = lax.axis_index("s")
        wid = cid * NS + sid
        base = sid * ROWS_LO

        @pl.when(sid < NS - 1)
        def _():
            pltpu.sync_copy(zeros_hbm.at[pl.ds(0, Z_LO)],
                            acc.at[pl.ds(sid * Z_LO, Z_LO)])

        @pl.when(sid == NS - 1)
        def _():
            pltpu.sync_copy(zeros_hbm.at[pl.ds(0, Z_HI)],
                            acc.at[pl.ds(15 * Z_LO, Z_HI)])

        plsc.subcore_barrier()

        pltpu.async_copy(src_hbm.at[wid, 0], sidx.at[0], sem_si.at[0])
        pltpu.async_copy(dst_hbm.at[wid, 0], didx.at[0], sem_di.at[0])
        pltpu.async_copy(src_hbm.at[wid, 1], sidx.at[1], sem_si.at[1])
        pltpu.async_copy(dst_hbm.at[wid, 1], didx.at[1], sem_di.at[1])
        pltpu.make_async_copy(src_hbm.at[wid, 0], sidx.at[0], sem_si.at[0]).wait()
        pltpu.async_copy(table_hbm.at[sidx.at[0]], rows.at[0], sem_g.at[0])

        def body(i, carry):
            ib = lax.rem(i, 2)
            nb = 1 - ib
            s_cur = lax.rem(i, 3)
            s_nxt = lax.rem(i + 1, 3)
            s_pre = lax.rem(i + 2, 3)

            @pl.when(i >= 1)
            def _():
                pltpu.make_async_copy(rows.at[nb], acc.at[didx.at[lax.rem(i - 1, 3)]],
                                      sem_s.at[nb]).wait()

            @pl.when(i + 2 < STEPS)
            def _():
                pltpu.async_copy(src_hbm.at[wid, i + 2], sidx.at[s_pre],
                                 sem_si.at[s_pre])
                pltpu.async_copy(dst_hbm.at[wid, i + 2], didx.at[s_pre],
                                 sem_di.at[s_pre])

            @pl.when(i + 1 < STEPS)
            def _():
                pltpu.make_async_copy(src_hbm.at[wid, i + 1], sidx.at[s_nxt],
                                      sem_si.at[s_nxt]).wait()
                pltpu.async_copy(table_hbm.at[sidx.at[s_nxt]], rows.at[nb],
                                 sem_g.at[nb])

            pltpu.make_async_copy(table_hbm.at[sidx.at[s_cur]], rows.at[ib],
                                  sem_g.at[ib]).wait()
            pltpu.make_async_copy(dst_hbm.at[wid, i], didx.at[s_cur],
                                  sem_di.at[s_cur]).wait()
            pltpu.async_copy(rows.at[ib], acc.at[didx.at[s_cur]], sem_s.at[ib],
                             add=True)
            return carry

        lax.fori_loop(0, STEPS, body, 0)
        last = lax.rem(STEPS - 1, 2)
        pltpu.make_async_copy(rows.at[last], acc.at[didx.at[lax.rem(STEPS - 1, 3)]],
                              sem_s.at[last]).wait()
        plsc.subcore_barrier()

        @pl.when(sid < NS - 1)
        def _():
            pltpu.sync_copy(acc.at[pl.ds(base, ROWS_LO)],
                            out_hbm.at[cid, pl.ds(base, ROWS_LO)])

        @pl.when(sid == NS - 1)
        def _():
            pltpu.sync_copy(acc.at[pl.ds(15 * ROWS_LO, ROWS_HI)],
                            out_hbm.at[cid, pl.ds(15 * ROWS_LO, ROWS_HI)])

    return k(src3, dst3, table, zeros)


def _scalar_pass(src4, dst4, t1d, zeros1):
    @functools.partial(
        pl.kernel,
        out_type=jax.ShapeDtypeStruct((NC, N), jnp.float32),
        mesh=_mesh,
        compiler_params=pltpu.CompilerParams(use_tc_tiling_on_sc=False),
        scratch_types=[
            pltpu.VMEM_SHARED((NPAD,), jnp.float32),
            pltpu.VMEM((STEPS_E, SUPER), jnp.int32),
            pltpu.VMEM((STEPS_E, SUPER), jnp.int32),
            pltpu.VMEM((2, SUPER), jnp.float32),
            pltpu.SemaphoreType.DMA((2,)),
            pltpu.SemaphoreType.DMA((2,)),
        ],
    )
    def k(src_hbm, dst_hbm, t_hbm, zeros_hbm, out_hbm,
          acc, sidx, didx, rows, sem_g, sem_s):
        cid = lax.axis_index("c")
        sid = lax.axis_index("s")
        wid = cid * NS + sid
        base = sid * ROWS_LO

        @pl.when(sid < NS - 1)
        def _():
            pltpu.sync_copy(zeros_hbm.at[pl.ds(0, Z_LO)],
                            acc.at[pl.ds(sid * Z_LO, Z_LO)])

        @pl.when(sid == NS - 1)
        def _():
            pltpu.sync_copy(zeros_hbm.at[pl.ds(0, Z_HI)],
                            acc.at[pl.ds(15 * Z_LO, Z_HI)])

        pltpu.sync_copy(src_hbm.at[wid], sidx)
        pltpu.sync_copy(dst_hbm.at[wid], didx)
        plsc.subcore_barrier()

        pltpu.async_copy(t_hbm.at[sidx.at[0]], rows.at[0], sem_g.at[0])

        def body(i, carry):
            ib = lax.rem(i, 2)
            nb = 1 - ib

            @pl.when(i >= 1)
            def _():
                pltpu.make_async_copy(rows.at[nb], acc.at[didx.at[i - 1]],
                                      sem_s.at[nb]).wait()

            @pl.when(i + 1 < STEPS_E)
            def _():
                pltpu.async_copy(t_hbm.at[sidx.at[i + 1]], rows.at[nb],
                                 sem_g.at[nb])

            pltpu.make_async_copy(t_hbm.at[sidx.at[i]], rows.at[ib],
                                  sem_g.at[ib]).wait()
            pltpu.async_copy(rows.at[ib], acc.at[didx.at[i]], sem_s.at[ib],
                             add=True)
            return carry

        lax.fori_loop(0, STEPS_E, body, 0)
        last = lax.rem(STEPS_E - 1, 2)
        pltpu.make_async_copy(rows.at[last], acc.at[didx.at[STEPS_E - 1]],
                              sem_s.at[last]).wait()
        plsc.subcore_barrier()

        @pl.when(sid < NS - 1)
        def _():
            pltpu.sync_copy(acc.at[pl.ds(base, ROWS_LO)],
                            out_hbm.at[cid, pl.ds(base, ROWS_LO)])

        @pl.when(sid == NS - 1)
        def _():
            pltpu.sync_copy(acc.at[pl.ds(15 * ROWS_LO, ROWS_HI)],
                            out_hbm.at[cid, pl.ds(15 * ROWS_LO, ROWS_HI)])

    return k(src4, dst4, t1d, zeros1)


def _degrees(src4, dst4, ones, zeros1):
    @functools.partial(
        pl.kernel,
        out_type=(jax.ShapeDtypeStruct((NC, N), jnp.float32),
                  jax.ShapeDtypeStruct((NC, N), jnp.float32)),
        mesh=_mesh,
        compiler_params=pltpu.CompilerParams(use_tc_tiling_on_sc=False),
        scratch_types=[
            pltpu.VMEM_SHARED((NPAD,), jnp.float32),
            pltpu.VMEM_SHARED((NPAD,), jnp.float32),
            pltpu.VMEM((STEPS_E, SUPER), jnp.int32),
            pltpu.VMEM((STEPS_E, SUPER), jnp.int32),
            pltpu.VMEM((SUPER,), jnp.float32),
            pltpu.SemaphoreType.DMA((2,)),
            pltpu.SemaphoreType.DMA((2,)),
        ],
    )
    def k(src_hbm, dst_hbm, ones_hbm, zeros_hbm, outs_hbm, outd_hbm,
          acc_s, acc_d, sidx, didx, ones_v, sem_s, sem_d):
        cid = lax.axis_index("c")
        sid = lax.axis_index("s")
        wid = cid * NS + sid
        base = sid * ROWS_LO

        pltpu.sync_copy(ones_hbm, ones_v)

        @pl.when(sid < NS - 1)
        def _():
            pltpu.sync_copy(zeros_hbm.at[pl.ds(0, Z_LO)],
                            acc_s.at[pl.ds(sid * Z_LO, Z_LO)])
            pltpu.sync_copy(zeros_hbm.at[pl.ds(0, Z_LO)],
                            acc_d.at[pl.ds(sid * Z_LO, Z_LO)])

        @pl.when(sid == NS - 1)
        def _():
            pltpu.sync_copy(zeros_hbm.at[pl.ds(0, Z_HI)],
                            acc_s.at[pl.ds(15 * Z_LO, Z_HI)])
            pltpu.sync_copy(zeros_hbm.at[pl.ds(0, Z_HI)],
                            acc_d.at[pl.ds(15 * Z_LO, Z_HI)])

        pltpu.sync_copy(src_hbm.at[wid], sidx)
        pltpu.sync_copy(dst_hbm.at[wid], didx)
        plsc.subcore_barrier()

        def body(i, carry):
            ib = lax.rem(i, 2)
            nb = 1 - ib

            @pl.when(i >= 1)
            def _():
                pltpu.make_async_copy(ones_v, acc_s.at[sidx.at[i - 1]],
                                      sem_s.at[nb]).wait()
                pltpu.make_async_copy(ones_v, acc_d.at[didx.at[i - 1]],
                                      sem_d.at[nb]).wait()

            pltpu.async_copy(ones_v, acc_s.at[sidx.at[i]], sem_s.at[ib], add=True)
            pltpu.async_copy(ones_v, acc_d.at[didx.at[i]], sem_d.at[ib], add=True)
            return carry

        lax.fori_loop(0, STEPS_E, body, 0)
        last = lax.rem(STEPS_E - 1, 2)
        pltpu.make_async_copy(ones_v, acc_s.at[sidx.at[STEPS_E - 1]],
                              sem_s.at[last]).wait()
        pltpu.make_async_copy(ones_v, acc_d.at[didx.at[STEPS_E - 1]],
                              sem_d.at[last]).wait()
        plsc.subcore_barrier()

        @pl.when(sid < NS - 1)
        def _():
            pltpu.sync_copy(acc_s.at[pl.ds(base, ROWS_LO)],
                            outs_hbm.at[cid, pl.ds(base, ROWS_LO)])
            pltpu.sync_copy(acc_d.at[pl.ds(base, ROWS_LO)],
                            outd_hbm.at[cid, pl.ds(base, ROWS_LO)])

        @pl.when(sid == NS - 1)
        def _():
            pltpu.sync_copy(acc_s.at[pl.ds(15 * ROWS_LO, ROWS_HI)],
                            outs_hbm.at[cid, pl.ds(15 * ROWS_LO, ROWS_HI)])
            pltpu.sync_copy(acc_d.at[pl.ds(15 * ROWS_LO, ROWS_HI)],
                            outd_hbm.at[cid, pl.ds(15 * ROWS_LO, ROWS_HI)])

    return k(src4, dst4, ones, zeros1)


def _tc_norms_g0(dpo, dpi, x, w0):
    def body(dpo_ref, dpi_ref, x_ref, w_ref, ns_ref, nd_ref, g0_ref):
        deg_o = dpo_ref[0] + dpo_ref[1]
        deg_i = dpi_ref[0] + dpi_ref[1]
        ns = lax.rsqrt(jnp.maximum(deg_o, 1.0))
        nd = lax.rsqrt(jnp.maximum(deg_i, 1.0))
        ns_ref[...] = ns
        nd_ref[...] = nd
        y = jnp.dot(x_ref[...], w_ref[...], preferred_element_type=jnp.float32)
        g0_ref[0:N, :] = y * ns
        g0_ref[N:NPAD, :] = jnp.zeros((DUMMY, HID), jnp.float32)

    return pl.pallas_call(
        body,
        out_shape=(jax.ShapeDtypeStruct((N, 1), jnp.float32),
                   jax.ShapeDtypeStruct((N, 1), jnp.float32),
                   jax.ShapeDtypeStruct((NPAD, HID), jnp.float32)),
    )(dpo, dpi, x, w0)


def _tc_layer(aggp, nd, ns, b, w_next):
    def body(p_ref, nd_ref, ns_ref, b_ref, w_ref, h_ref, g_ref):
        h = (p_ref[0] + p_ref[1]) * nd_ref[...] + b_ref[...]
        h_ref[...] = h
        g_ref[0:N, :] = jnp.dot(h, w_ref[...], preferred_element_type=jnp.float32) * ns_ref[...]
        g_ref[N:NPAD, :] = jnp.zeros((DUMMY, HID), jnp.float32)

    return pl.pallas_call(
        body,
        out_shape=(jax.ShapeDtypeStruct((N, HID), jnp.float32),
                   jax.ShapeDtypeStruct((NPAD, HID), jnp.float32)),
    )(aggp, nd, ns, b, w_next)


def _tc_score_prep(aggp, nd, ns, b2, h1, h2, ws):
    def body(p_ref, nd_ref, ns_ref, b_ref, h1_ref, h2_ref, ws_ref, h3_ref, t_ref):
        h3 = (p_ref[0] + p_ref[1]) * nd_ref[...] + b_ref[...]
        h3_ref[...] = h3
        t = (jnp.dot(h1_ref[...], ws_ref[0:HID], preferred_element_type=jnp.float32)
             + jnp.dot(h2_ref[...], ws_ref[HID:2 * HID], preferred_element_type=jnp.float32)
             + jnp.dot(h3, ws_ref[2 * HID:3 * HID], preferred_element_type=jnp.float32))
        t_ref[...] = t * ns_ref[...]

    return pl.pallas_call(
        body,
        out_shape=(jax.ShapeDtypeStruct((N, HID), jnp.float32),
                   jax.ShapeDtypeStruct((N, 1), jnp.float32)),
    )(aggp, nd, ns, b2, h1, h2, ws)


def _tc_final(spr, spc, nd_row, nd, bs, h1, h2, h3, seq,
              wl1, bl1, wl2, bl2, wl3, bl3):
    def body(spr_ref, spc_ref, ndr_ref, nd_ref, bs_ref,
             h1_ref, h2_ref, h3_ref, seq_ref,
             wl1_ref, bl1_ref, wl2_ref, bl2_ref, wl3_ref, bl3_ref, out_ref):
        min_i32 = jnp.int32(-2147483648)
        mask7f = jnp.int32(0x7FFFFFFF)

        s_row = (spr_ref[0] + spr_ref[1]) * ndr_ref[...] + bs_ref[...]
        bits_r = lax.bitcast_convert_type(s_row, jnp.int32)
        key_r = bits_r ^ (mask7f & (bits_r >> 31))

        def bis_a(i, acc):
            cand = acc | (jnp.int32(1) << (31 - i))
            cnt = jnp.sum((key_r >= (cand ^ min_i32)).astype(jnp.int32))
            return jnp.where(cnt >= K, cand, acc)

        tau_u = lax.fori_loop(0, 32, bis_a, jnp.int32(0))
        tau_s = tau_u ^ min_i32

        cnt_gt = jnp.sum((key_r > tau_s).astype(jnp.int32))
        r = K - cnt_gt
        eq_r = key_r == tau_s
        idx_r = lax.broadcasted_iota(jnp.int32, (1, N), 1)

        def bis_b(i, acc):
            cand = acc | (jnp.int32(1) << (13 - i))
            f = jnp.sum((eq_r & (idx_r < cand)).astype(jnp.int32))
            return jnp.where(f <= r, cand, acc)

        m = lax.fori_loop(0, 14, bis_b, jnp.int32(0))

        s_col = (spc_ref[0] + spc_ref[1]) * nd_ref[...] + bs_ref[0, 0]
        bits_c = lax.bitcast_convert_type(s_col, jnp.int32)
        key_c = bits_c ^ (mask7f & (bits_c >> 31))
        idx_c = lax.broadcasted_iota(jnp.int32, (N, 1), 0)
        sel = (key_c > tau_s) | ((key_c == tau_s) & (idx_c < m))

        w1 = jnp.tanh(s_col) * sel.astype(jnp.float32)

        cat = jnp.concatenate([h1_ref[...], h2_ref[...], h3_ref[...]], axis=1)
        pooled = cat * w1
        avg = jnp.sum(pooled, axis=0, keepdims=True) * (1.0 / K)
        neg = jnp.float32(-jnp.inf)
        mx = jnp.max(jnp.where(sel, pooled, neg), axis=0, keepdims=True)

        feat = jnp.concatenate([avg, mx, seq_ref[...]], axis=1)
        a1 = jnp.maximum(
            jnp.dot(feat, wl1_ref[...], preferred_element_type=jnp.float32)
            + bl1_ref[...], 0.0)
        a2 = jnp.maximum(
            jnp.dot(a1, wl2_ref[...], preferred_element_type=jnp.float32)
            + bl2_ref[...], 0.0)
        out_ref[...] = (jnp.dot(a2, wl3_ref[...], preferred_element_type=jnp.float32)
                        + bl3_ref[...])

    return pl.pallas_call(
        body,
        out_shape=jax.ShapeDtypeStruct((1, 128), jnp.float32),
    )(spr, spc, nd_row, nd, bs, h1, h2, h3, seq, wl1, bl1, wl2, bl2, wl3, bl3)


def kernel(x, edge_index, sequence_feature, W0, b0, W1, b1, W2, b2, Ws, bs,
           Wl1, bl1, Wl2, bl2, Wl3, bl3):
    pad = (N + (jnp.arange(EPAD - E, dtype=jnp.int32) % DUMMY)).astype(jnp.int32)
    src_flat = jnp.concatenate([edge_index[0], pad])
    dst_flat = jnp.concatenate([edge_index[1], pad])
    src = src_flat.reshape(NW, STEPS, CHUNK)
    dst = dst_flat.reshape(NW, STEPS, CHUNK)
    src_e = src_flat.reshape(NW, STEPS_E, SUPER)
    dst_e = dst_flat.reshape(NW, STEPS_E, SUPER)

    zeros128 = jnp.zeros((ZROWS, HID), jnp.float32)
    zeros1 = jnp.zeros((ZROWS,), jnp.float32)
    ones_e = jnp.ones((SUPER,), jnp.float32)

    dpo, dpi = _degrees(src_e, dst_e, ones_e, zeros1)

    ns, nd, g0 = _tc_norms_g0(dpo.reshape(NC, N, 1), dpi.reshape(NC, N, 1), x, W0)

    p0 = _mp_pass(src, dst, g0, zeros128)
    h1, g1 = _tc_layer(p0, nd, ns, b0.reshape(1, HID), W1)
    p1 = _mp_pass(src, dst, g1, zeros128)
    h2, g2 = _tc_layer(p1, nd, ns, b1.reshape(1, HID), W2)
    p2 = _mp_pass(src, dst, g2, zeros128)
    h3, t = _tc_score_prep(p2, nd, ns, b2.reshape(1, HID), h1, h2, Ws)

    t1d = jnp.concatenate([t.reshape(N), jnp.zeros((DUMMY,), jnp.float32)])
    sp = _scalar_pass(src_e, dst_e, t1d, zeros1)

    return _tc_final(sp.reshape(NC, 1, N), sp.reshape(NC, N, 1),
                     nd.reshape(1, N), nd, bs.reshape(1, 1),
                     h1, h2, h3, sequence_feature,
                     Wl1, bl1, Wl2, bl2, Wl3, bl3)
